# baseline (speedup 1.0000x reference)
import functools

import jax
import jax.numpy as jnp
from jax import lax
from jax.experimental import pallas as pl
from jax.experimental.pallas import tpu as pltpu
from jax.experimental.pallas import tpu_sc as plsc

DIM = 128
NC = 2
NS = 16
NW = NC * NS
CHUNK = 128
N_CHUNKS = 25
B_PER_W = CHUNK * N_CHUNKS
BP = NW * B_PER_W


def _emb_kernel(table_hbm, idx_hbm, out_hbm, idx_v, rows_v, gsem):
    wid = lax.axis_index("s") * NC + lax.axis_index("c")
    base = wid * B_PER_W
    pltpu.sync_copy(idx_hbm.at[pl.ds(base, B_PER_W)], idx_v)
    for g in range(N_CHUNKS):
        buf = rows_v.at[g % 2]
        pltpu.async_copy(
            table_hbm.at[idx_v.at[pl.ds(g * CHUNK, CHUNK)]], buf, gsem
        ).wait()
        pltpu.sync_copy(buf, out_hbm.at[pl.ds(base + g * CHUNK, CHUNK), :])


@functools.partial(jax.jit, static_argnums=())
def _gather(table, idx2):
    mesh = plsc.VectorSubcoreMesh(core_axis_name="c", subcore_axis_name="s")
    f = functools.partial(
        pl.kernel,
        mesh=mesh,
        out_type=jax.ShapeDtypeStruct((BP, DIM), jnp.float32),
        scratch_types=[
            pltpu.VMEM((B_PER_W,), jnp.int32),
            pltpu.VMEM((2, CHUNK, DIM), jnp.float32),
            pltpu.SemaphoreType.DMA,
        ],
    )(_emb_kernel)
    return f(table, idx2)


def kernel(node_type, table):
    t = table.at[0].set(0.0)
    b = node_type.shape[0]
    idx = jnp.pad(node_type, (0, BP - b))
    out = _gather(t, idx)
    return out[:b]


# speedup vs baseline: 1.5960x; 1.5960x over previous
import functools

import jax
import jax.numpy as jnp
from jax import lax
from jax.experimental import pallas as pl
from jax.experimental.pallas import tpu as pltpu
from jax.experimental.pallas import tpu_sc as plsc

DIM = 128
NC = 2
NS = 16
NW = NC * NS
N = 100000
SPAN = 3128
LAST_BASE = N - SPAN
CHUNK = 256
SIZES = [CHUNK] * 12 + [56]
OFFS = [sum(SIZES[:i]) for i in range(len(SIZES))]
NCH = len(SIZES)


def _emb_kernel(table_hbm, idx_hbm, out_hbm,
                idx_v, buf0, buf1, gs0, gs1, ws0, ws1):
    wid = lax.axis_index("s") * NC + lax.axis_index("c")
    base = lax.min(wid * SPAN, LAST_BASE)
    bufs = (buf0, buf1)
    gsems = (gs0, gs1)
    wsems = (ws0, ws1)
    pltpu.sync_copy(idx_hbm.at[pl.ds(base, SPAN)], idx_v)

    def start_gather(g):
        sz = SIZES[g]
        dst = bufs[g % 2].at[pl.ds(0, sz)] if sz != CHUNK else bufs[g % 2]
        return pltpu.async_copy(
            table_hbm.at[idx_v.at[pl.ds(OFFS[g], sz)]], dst, gsems[g % 2])

    def start_write(g):
        sz = SIZES[g]
        src = bufs[g % 2].at[pl.ds(0, sz)] if sz != CHUNK else bufs[g % 2]
        return pltpu.async_copy(
            src, out_hbm.at[pl.ds(base + OFFS[g], sz), :], wsems[g % 2])

    gathers = [None] * NCH
    writes = [None] * NCH
    gathers[0] = start_gather(0)
    for g in range(NCH):
        nxt = g + 1
        if nxt < NCH:
            if nxt >= 2:
                writes[nxt - 2].wait()
            gathers[nxt] = start_gather(nxt)
        gathers[g].wait()
        writes[g] = start_write(g)
    writes[NCH - 2].wait()
    writes[NCH - 1].wait()


@jax.jit
def _gather(table, idx):
    mesh = plsc.VectorSubcoreMesh(core_axis_name="c", subcore_axis_name="s")
    f = functools.partial(
        pl.kernel,
        mesh=mesh,
        out_type=jax.ShapeDtypeStruct((N, DIM), jnp.float32),
        scratch_types=[
            pltpu.VMEM((SPAN,), jnp.int32),
            pltpu.VMEM((CHUNK, DIM), jnp.float32),
            pltpu.VMEM((CHUNK, DIM), jnp.float32),
            pltpu.SemaphoreType.DMA,
            pltpu.SemaphoreType.DMA,
            pltpu.SemaphoreType.DMA,
            pltpu.SemaphoreType.DMA,
        ],
    )(_emb_kernel)
    return f(table, idx)


def kernel(node_type, table):
    t = table.at[0].set(0.0)
    return _gather(t, node_type)


# speedup vs baseline: 6.2078x; 3.8896x over previous
import functools

import jax
import jax.numpy as jnp
from jax import lax
from jax.experimental import pallas as pl
from jax.experimental.pallas import tpu as pltpu
from jax.experimental.pallas import tpu_sc as plsc

DIM = 128
NC = 2
NS = 16
NW = NC * NS
N = 100000
NROWS = 100
SPAN = 3128
LAST_BASE = N - SPAN
CHUNK = 256
SIZES = [CHUNK] * 12 + [56]
OFFS = [sum(SIZES[:i]) for i in range(len(SIZES))]
NCH = len(SIZES)


def _emb_kernel(table_hbm, idx_hbm, out_hbm,
                table_sh, idx_v, buf0, buf1, gs0, gs1, ws0, ws1):
    sid = lax.axis_index("s")
    wid = sid * NC + lax.axis_index("c")
    base = lax.min(wid * SPAN, LAST_BASE)
    bufs = (buf0, buf1)
    gsems = (gs0, gs1)
    wsems = (ws0, ws1)

    @pl.when(sid == 0)
    def _stage():
        pltpu.sync_copy(table_hbm, table_sh)
        zeros = jnp.zeros((16,), jnp.float32)
        for j in range(DIM // 16):
            buf0[0, pl.ds(j * 16, 16)] = zeros
        pltpu.sync_copy(buf0.at[pl.ds(0, 1)], table_sh.at[pl.ds(0, 1)])

    plsc.subcore_barrier()
    pltpu.sync_copy(idx_hbm.at[pl.ds(base, SPAN)], idx_v)

    def start_gather(g):
        sz = SIZES[g]
        dst = bufs[g % 2].at[pl.ds(0, sz)] if sz != CHUNK else bufs[g % 2]
        return pltpu.async_copy(
            table_sh.at[idx_v.at[pl.ds(OFFS[g], sz)]], dst, gsems[g % 2])

    def start_write(g):
        sz = SIZES[g]
        src = bufs[g % 2].at[pl.ds(0, sz)] if sz != CHUNK else bufs[g % 2]
        return pltpu.async_copy(
            src, out_hbm.at[pl.ds(base + OFFS[g], sz), :], wsems[g % 2])

    gathers = [None] * NCH
    writes = [None] * NCH
    gathers[0] = start_gather(0)
    for g in range(NCH):
        nxt = g + 1
        if nxt < NCH:
            if nxt >= 2:
                writes[nxt - 2].wait()
            gathers[nxt] = start_gather(nxt)
        gathers[g].wait()
        writes[g] = start_write(g)
    writes[NCH - 2].wait()
    writes[NCH - 1].wait()


@jax.jit
def _gather(table, idx):
    mesh = plsc.VectorSubcoreMesh(core_axis_name="c", subcore_axis_name="s")
    f = functools.partial(
        pl.kernel,
        mesh=mesh,
        out_type=jax.ShapeDtypeStruct((N, DIM), jnp.float32),
        scratch_types=[
            pltpu.VMEM_SHARED((NROWS, DIM), jnp.float32),
            pltpu.VMEM((SPAN,), jnp.int32),
            pltpu.VMEM((CHUNK, DIM), jnp.float32),
            pltpu.VMEM((CHUNK, DIM), jnp.float32),
            pltpu.SemaphoreType.DMA,
            pltpu.SemaphoreType.DMA,
            pltpu.SemaphoreType.DMA,
            pltpu.SemaphoreType.DMA,
        ],
    )(_emb_kernel)
    return f(table, idx)


def kernel(node_type, table):
    return _gather(table, node_type)
